# baseline (speedup 1.0000x reference)
import functools

import jax
import jax.numpy as jnp
from jax import lax
from jax.experimental import pallas as pl
from jax.experimental.pallas import tpu as pltpu
from jax.experimental.pallas import tpu_sc as plsc

N = 10000
NP = 10240
E = 320000
H = 128
NG = 16
NVT = 4
DM = 256
DI = 512
DS = 128
DTR = 16
GGC = 8
NL = 2

NC = 2
NSUB = 16
NW = NC * NSUB
CH = 64
NCH = 160
EPW = NCH * CH
EPAD = EPW * NW
NPAD = NP
SLAB = NPAD // NSUB
PH = 4
CPP = NCH // PH

FB = 1024
GRID = NP // FB


def _sc_agg_body(h_hbm, srcs, dsts, zrows, out, src_v, dst_v, r0, r1, r2, r3,
                 a_sh, sem0, sem1, sem2, sem3, ssem0, ssem1, ssem2, ssem3):
    c = lax.axis_index("c")
    s = lax.axis_index("s")
    wid = c * NSUB + s

    pltpu.sync_copy(zrows, a_sh.at[pl.ds(s * SLAB, SLAB)])
    plsc.subcore_barrier()

    def gather(j, buf, sem):
        return pltpu.async_copy(h_hbm.at[src_v.at[j]], buf, sem)

    def wait(j, buf, sem):
        pltpu.make_async_copy(h_hbm.at[src_v.at[j]], buf, sem).wait()

    def scat_start(buf, j, sem):
        return pltpu.async_copy(buf, a_sh.at[dst_v.at[j]], sem, add=True)

    def scat_wait(buf, j, sem):
        pltpu.make_async_copy(buf, a_sh.at[dst_v.at[j]], sem).wait()

    bufs = (r0, r1, r2, r3)
    gsems = (sem0, sem1, sem2, sem3)
    ssems = (ssem0, ssem1, ssem2, ssem3)
    for p in range(PH):
        pltpu.sync_copy(srcs.at[wid, pl.ds(p * CPP, CPP)], src_v)
        pltpu.sync_copy(dsts.at[wid, pl.ds(p * CPP, CPP)], dst_v)
        for k in range(4):
            gather(k, bufs[k], gsems[k])

        def body(g, carry):
            for k in range(4):
                j = 4 * g + k
                wait(j, bufs[k], gsems[k])
                scat_start(bufs[k], j, ssems[k])

            @pl.when(g < CPP // 4 - 1)
            def _():
                for k in range(4):
                    j = 4 * g + k
                    scat_wait(bufs[k], j, ssems[k])
                    gather(j + 4, bufs[k], gsems[k])
            return carry

        lax.fori_loop(0, CPP // 4, body, 0)
        for k in range(4):
            scat_wait(bufs[k], CPP - 4 + k, ssems[k])
    plsc.subcore_barrier()

    pltpu.sync_copy(a_sh.at[pl.ds(s * SLAB, SLAB)],
                    out.at[c, pl.ds(s * SLAB, SLAB)])


@functools.cache
def _make_sc_agg():
    return functools.partial(
        pl.kernel,
        out_type=jax.ShapeDtypeStruct((NC, NPAD, H), jnp.float32),
        mesh=plsc.VectorSubcoreMesh(core_axis_name="c", subcore_axis_name="s",
                                    num_cores=NC, num_subcores=NSUB),
        scratch_types=[
            pltpu.VMEM((CPP, CH), jnp.int32),
            pltpu.VMEM((CPP, CH), jnp.int32),
            pltpu.VMEM((CH, H), jnp.float32),
            pltpu.VMEM((CH, H), jnp.float32),
            pltpu.VMEM((CH, H), jnp.float32),
            pltpu.VMEM((CH, H), jnp.float32),
            pltpu.VMEM_SHARED((NPAD, H), jnp.float32),
        ] + [pltpu.SemaphoreType.DMA] * 8,
    )(_sc_agg_body)


def _sc_agg(h, src_p, dst_p, zrows):
    return _make_sc_agg()(h, src_p, dst_p, zrows)


def _ln(h, g, b):
    mu = jnp.mean(h, -1, keepdims=True)
    v = jnp.mean((h - mu) ** 2, -1, keepdims=True)
    return (h - mu) * jax.lax.rsqrt(v + 1e-5) * g + b


def _dot(a, b):
    return jnp.dot(a, b, preferred_element_type=jnp.float32)


def _wg_body(g_ref, wih_t_ref, out_ref):
    wih_t = wih_t_ref[...]
    for l in range(GGC):
        out_ref[l] = _dot(g_ref[l], wih_t)


def _enc_body(x_ref, w0t, b0, g0, be0, w1t, b1, out_ref):
    h = _dot(x_ref[...], w0t[...]) + b0[...]
    h = jax.nn.relu(_ln(h, g0[...], be0[...]))
    out_ref[...] = _dot(h, w1t[...]) + b1[...]


def _gru_body(h_ref, a2_ref, wg, whht, bih, bhh, out_ref):
    a = a2_ref[0] + a2_ref[1]
    gi = _dot(a, wg[0]) + bih[...]
    gh = _dot(h_ref[...], whht[...]) + bhh[...]
    r = jax.nn.sigmoid(gi[:, :H] + gh[:, :H])
    z = jax.nn.sigmoid(gi[:, H:2 * H] + gh[:, H:2 * H])
    n = jnp.tanh(gi[:, 2 * H:] + r * gh[:, 2 * H:])
    out_ref[...] = (1.0 - z) * n + z * h_ref[...]


def _score_body(x_ref, pe_ref, thx, thp, out_ref):
    out_ref[...] = (
        lax.dot_general(thx[...], x_ref[...], (((1,), (1,)), ((), ())),
                        preferred_element_type=jnp.float32)
        + lax.dot_general(thp[...], pe_ref[...], (((1,), (1,)), ((), ())),
                          preferred_element_type=jnp.float32))


def _pool_body(s_ref, batch_ref, idx_ref, scale_ref):
    s = s_ref[...]
    brow = batch_ref[...]
    counts = []
    for b in range(NG):
        m = (brow == b).astype(jnp.float32)
        counts.append(jnp.sum(m, axis=1, keepdims=True))
    mxn = counts[0]
    for b in range(1, NG):
        mxn = jnp.maximum(mxn, counts[b])
    iota = lax.broadcasted_iota(jnp.int32, (NVT, NP), 1)
    neg = jnp.float32(-3e38)
    for b in range(NG):
        m = brow == b
        vals = jnp.where(m, s, neg)
        mx = jnp.max(vals, axis=1, keepdims=True)
        eq = vals == mx
        idx = jnp.min(jnp.where(eq, iota, jnp.int32(NP)), axis=1, keepdims=True)
        idx = jnp.minimum(idx, N - 1)
        keep = (mx >= 0.0) | (counts[b] == mxn)
        scale = keep.astype(jnp.float32) / mxn
        idx_ref[:, pl.ds(b, 1)] = idx
        scale_ref[:, pl.ds(b, 1)] = scale


def _tok_body(x_ref, pe_ref, idx_ref, scale_ref, m_int, cwt, convb, dtw, bw,
              cw_, dtwt, dtb, alog, md, moutt, lng, lnb, mwbt, mergeb,
              out_ref):
    rows = []
    for b in range(NG):
        for k in range(NVT):
            i = idx_ref[k, b]
            sc = scale_ref[k, b]
            row = jnp.concatenate(
                [x_ref[pl.ds(i, 1), :], pe_ref[pl.ds(i, 1), :]], axis=1)
            rows.append(row * sc)
    tokens = jnp.concatenate(rows, axis=0)

    xz = _dot(tokens, m_int[...])
    xi = xz[:, :DI]
    z3 = xz.reshape(NG, NVT, 2 * DI)[:, NVT - 1, DI:]

    xi4 = xi.reshape(NG, NVT, DI)
    cw = cwt[...]
    convs = []
    for t in range(NVT):
        acc = None
        for tau in range(t + 1):
            term = xi4[:, tau, :] * cw[3 - t + tau:4 - t + tau, :]
            acc = term if acc is None else acc + term
        convs.append(acc + convb[...])
    xs = jnp.concatenate([c.reshape(NG, 1, DI) for c in convs], axis=1)
    xs = xs * jax.nn.sigmoid(xs)
    xsf = xs.reshape(NG * NVT, DI)

    dt_in = _dot(xsf, dtw[...])
    bc = _dot(xsf, bw[...]).reshape(NG, NVT, DS)
    cc3 = _dot(xsf, cw_[...]).reshape(NG, NVT, DS)[:, NVT - 1, :]
    dt = jax.nn.softplus(_dot(dt_in, dtwt[...]) + dtb[...])
    dt4 = dt.reshape(NG, NVT, DI)

    a_full = -jnp.exp(alog[...])
    y_parts = []
    for dc in range(DI // DS):
        lo, hi = dc * DS, (dc + 1) * DS
        a_c = a_full[lo:hi, :]
        h = jnp.zeros((NG, DS, DS), jnp.float32)
        for t in range(NVT):
            dtc = dt4[:, t, lo:hi]
            xic = xs[:, t, lo:hi]
            da = jnp.exp(dtc[:, :, None] * a_c[None, :, :])
            dbx = (dtc * xic)[:, :, None] * bc[:, t, :][:, None, :]
            h = da * h + dbx
        y_parts.append(jnp.sum(h * cc3[:, None, :], axis=2))
    y3 = jnp.concatenate(y_parts, axis=1)
    xi3 = xs[:, NVT - 1, :]
    y3 = y3 + md[...] * xi3
    y3 = y3 * (z3 * jax.nn.sigmoid(z3))
    xm = _dot(y3, moutt[...])
    xm = _ln(xm, lng[...], lnb[...])
    out_ref[...] = _dot(xm, mwbt[...]) + mergeb[...]


def _merge_body(x1_ref, bc_ref, t16_ref, mwat, out_ref):
    oh = (bc_ref[...] == lax.broadcasted_iota(jnp.int32, (1, NG), 1))
    out_ref[...] = (_dot(x1_ref[...], mwat[...])
                    + _dot(oh.astype(jnp.float32), t16_ref[...]))


def _final_body(x_ref, a2_ref, wot, bo, bc_ref, out_ref):
    i = pl.program_id(0)
    y = _dot(x_ref[...] + a2_ref[0] + a2_ref[1], wot[...]) + bo[...]
    oh = (bc_ref[...] == lax.broadcasted_iota(jnp.int32, (1, NG), 1))
    part = lax.dot_general(oh.astype(jnp.float32), y, (((0,), (0,)), ((), ())),
                           preferred_element_type=jnp.float32)

    @pl.when(i == 0)
    def _():
        out_ref[...] = jnp.zeros_like(out_ref)

    out_ref[...] += part


def _dec_body(s_ref, w0t, b0, g0, be0, w1t, b1, out_ref):
    h = _dot(s_ref[...], w0t[...]) + b0[...]
    h = jax.nn.relu(_ln(h, g0[...], be0[...]))
    out_ref[...] = _dot(h, w1t[...]) + b1[...]


def _row_spec(r, c):
    return pl.BlockSpec((r, c), lambda i: (i, 0))


def _full_spec(shape):
    nd = len(shape)
    return pl.BlockSpec(shape, lambda i, _n=nd: (0,) * _n)


def _call_rows(body, out_shape, row_ins, full_ins, out_full=False):
    in_specs = ([_row_spec(FB, c) for _, c in row_ins]
                + [_full_spec(a.shape) for a in full_ins])
    if out_full:
        out_spec = _full_spec(out_shape)
    else:
        out_spec = _row_spec(FB, out_shape[-1])
    return pl.pallas_call(
        body,
        grid=(GRID,),
        in_specs=in_specs,
        out_specs=out_spec,
        out_shape=jax.ShapeDtypeStruct(out_shape, jnp.float32),
    )(*[a for a, _ in row_ins], *full_ins)


def kernel(x, edge_index, batch, params):
    p = params
    f32 = jnp.float32

    perm = jnp.argsort(edge_index[0])
    src = edge_index[0][perm]
    dst = edge_index[1][perm]
    src_p = jnp.concatenate(
        [src, jnp.zeros((EPAD - E,), jnp.int32)]).reshape(NW, NCH, CH)
    dst_p = jnp.concatenate(
        [dst, jnp.full((EPAD - E,), NP - 1, jnp.int32)]).reshape(NW, NCH, CH)
    zrows = jnp.zeros((SLAB, H), f32)
    x = jnp.concatenate([x, jnp.zeros((NP - N, x.shape[1]), f32)])
    batch_p = jnp.concatenate([batch, jnp.full((NP - N,), NG, jnp.int32)])
    bcol = batch_p.reshape(NP, 1)
    brow = batch_p.reshape(1, NP)

    def row(v):
        return v.reshape(1, -1)

    wg = pl.pallas_call(
        _wg_body,
        out_shape=jax.ShapeDtypeStruct((GGC, H, 3 * H), f32),
    )(p['ggc_w'], p['gru_Wih'].T)

    x1 = _call_rows(
        _enc_body, (NP, H), [(x, H)],
        [p['enc_W0'].T, row(p['enc_b0']), row(p['enc_g0']), row(p['enc_be0']),
         p['enc_W1'].T, row(p['enc_b1'])])

    whht = p['gru_Whh'].T
    bih = row(p['gru_bih'])
    bhh = row(p['gru_bhh'])
    h = x1
    for l in range(GGC):
        a2 = _sc_agg(h, src_p, dst_p, zrows)
        h = pl.pallas_call(
            _gru_body,
            grid=(GRID,),
            in_specs=[
                _row_spec(FB, H),
                pl.BlockSpec((NC, FB, H), lambda i: (0, i, 0)),
                pl.BlockSpec((1, H, 3 * H), lambda i, _l=l: (_l, 0, 0)),
                _full_spec(whht.shape),
                _full_spec(bih.shape),
                _full_spec(bhh.shape),
            ],
            out_specs=_row_spec(FB, H),
            out_shape=jax.ShapeDtypeStruct((NP, H), f32),
        )(h, a2, wg, whht, bih, bhh)
    pe = h

    xc = x1
    for i in range(NL):
        thx = p['theta'][i][:, :H]
        thp = p['theta'][i][:, H:]
        s_t = pl.pallas_call(
            _score_body,
            grid=(GRID,),
            in_specs=[_row_spec(FB, H), _row_spec(FB, H),
                      _full_spec(thx.shape), _full_spec(thp.shape)],
            out_specs=pl.BlockSpec((NVT, FB), lambda i: (0, i)),
            out_shape=jax.ShapeDtypeStruct((NVT, NP), f32),
        )(xc, pe, thx, thp)

        idxs, scales = pl.pallas_call(
            _pool_body,
            out_shape=(jax.ShapeDtypeStruct((NVT, NG), jnp.int32),
                       jax.ShapeDtypeStruct((NVT, NG), f32)),
        )(s_t, brow)

        mw = [p['m_in'][i].T, p['m_conv_w'][i].T, row(p['m_conv_b'][i]),
              p['m_xproj'][i][:DTR].T, p['m_xproj'][i][DTR:DTR + DS].T,
              p['m_xproj'][i][DTR + DS:].T, p['m_dt_w'][i].T,
              row(p['m_dt_b'][i]), p['m_Alog'][i], row(p['m_D'][i]),
              p['m_out'][i].T, row(p['ln_m_g']), row(p['ln_m_b']),
              p['merge_W'][i][:, H:].T, row(p['merge_b'][i])]

        def _fs(a):
            return pl.BlockSpec(a.shape, lambda *_, _n=a.ndim: (0,) * _n)

        t16 = pl.pallas_call(
            _tok_body,
            in_specs=([_fs(xc), _fs(pe),
                       pl.BlockSpec(memory_space=pltpu.SMEM),
                       pl.BlockSpec(memory_space=pltpu.SMEM)]
                      + [_fs(a) for a in mw]),
            out_specs=pl.BlockSpec((NG, H), lambda *_: (0, 0)),
            out_shape=jax.ShapeDtypeStruct((NG, H), f32),
        )(xc, pe, idxs, scales, *mw)

        xc = _call_rows(
            _merge_body, (NP, H), [(x1, H), (bcol, 1)],
            [t16, p['merge_W'][i][:, :H].T])

    a2 = _sc_agg(xc, src_p, dst_p, zrows)
    seg = pl.pallas_call(
        _final_body,
        grid=(GRID,),
        in_specs=[
            _row_spec(FB, H),
            pl.BlockSpec((NC, FB, H), lambda i: (0, i, 0)),
            _full_spec((H, H)),
            _full_spec((1, H)),
            _row_spec(FB, 1),
        ],
        out_specs=pl.BlockSpec((NG, H), lambda i: (0, 0)),
        out_shape=jax.ShapeDtypeStruct((NG, H), f32),
    )(xc, a2, p['outgin_W'].T, row(p['outgin_b']), bcol)

    out = pl.pallas_call(
        _dec_body,
        out_shape=jax.ShapeDtypeStruct((NG, H), f32),
    )(seg, p['dec_W0'].T, row(p['dec_b0']), row(p['dec_g0']),
      row(p['dec_be0']), p['dec_W1'].T, row(p['dec_b1']))
    return out


# speedup vs baseline: 1.2109x; 1.2109x over previous
import functools

import jax
import jax.numpy as jnp
from jax import lax
from jax.experimental import pallas as pl
from jax.experimental.pallas import tpu as pltpu
from jax.experimental.pallas import tpu_sc as plsc

N = 10000
NP = 10240
E = 320000
H = 128
NG = 16
NVT = 4
DM = 256
DI = 512
DS = 128
DTR = 16
GGC = 8
NL = 2

NC = 2
NSUB = 16
NW = NC * NSUB
CH = 64
NCH = 160
EPW = NCH * CH
EPAD = EPW * NW
NPAD = NP
SLAB = NPAD // NSUB
PH = 4
CPP = NCH // PH

FB = 1024
GRID = NP // FB


def _sc_agg_body(h_hbm, srcs, dsts, zrows, out, src_v, dst_v, r0, r1, r2, r3,
                 a_sh, sem0, sem1, sem2, sem3, ssem0, ssem1, ssem2, ssem3):
    c = lax.axis_index("c")
    s = lax.axis_index("s")
    wid = c * NSUB + s

    pltpu.sync_copy(zrows, a_sh.at[pl.ds(s * SLAB, SLAB)])
    plsc.subcore_barrier()

    def gather(j, buf, sem):
        return pltpu.async_copy(h_hbm.at[src_v.at[j]], buf, sem)

    def wait(j, buf, sem):
        pltpu.make_async_copy(h_hbm.at[src_v.at[j]], buf, sem).wait()

    def scat_start(buf, j, sem):
        return pltpu.async_copy(buf, a_sh.at[dst_v.at[j]], sem, add=True)

    def scat_wait(buf, j, sem):
        pltpu.make_async_copy(buf, a_sh.at[dst_v.at[j]], sem).wait()

    bufs = (r0, r1, r2, r3)
    gsems = (sem0, sem1, sem2, sem3)
    ssems = (ssem0, ssem1, ssem2, ssem3)
    for p in range(PH):
        pltpu.sync_copy(srcs.at[wid, pl.ds(p * CPP, CPP)], src_v)
        pltpu.sync_copy(dsts.at[wid, pl.ds(p * CPP, CPP)], dst_v)
        for k in range(4):
            gather(k, bufs[k], gsems[k])

        def body(g, carry):
            for k in range(4):
                j = 4 * g + k
                wait(j, bufs[k], gsems[k])
                scat_start(bufs[k], j, ssems[k])

            @pl.when(g < CPP // 4 - 1)
            def _():
                for k in range(4):
                    j = 4 * g + k
                    scat_wait(bufs[k], j, ssems[k])
                    gather(j + 4, bufs[k], gsems[k])
            return carry

        lax.fori_loop(0, CPP // 4, body, 0)
        for k in range(4):
            scat_wait(bufs[k], CPP - 4 + k, ssems[k])
    plsc.subcore_barrier()

    pltpu.sync_copy(a_sh.at[pl.ds(s * SLAB, SLAB)],
                    out.at[c, pl.ds(s * SLAB, SLAB)])


@functools.cache
def _make_sc_agg():
    return functools.partial(
        pl.kernel,
        out_type=jax.ShapeDtypeStruct((NC, NPAD, H), jnp.float32),
        mesh=plsc.VectorSubcoreMesh(core_axis_name="c", subcore_axis_name="s",
                                    num_cores=NC, num_subcores=NSUB),
        scratch_types=[
            pltpu.VMEM((CPP, CH), jnp.int32),
            pltpu.VMEM((CPP, CH), jnp.int32),
            pltpu.VMEM((CH, H), jnp.float32),
            pltpu.VMEM((CH, H), jnp.float32),
            pltpu.VMEM((CH, H), jnp.float32),
            pltpu.VMEM((CH, H), jnp.float32),
            pltpu.VMEM_SHARED((NPAD, H), jnp.float32),
        ] + [pltpu.SemaphoreType.DMA] * 8,
    )(_sc_agg_body)


def _sc_agg(h, src_p, dst_p, zrows):
    return _make_sc_agg()(h, src_p, dst_p, zrows)


def _ln(h, g, b):
    mu = jnp.mean(h, -1, keepdims=True)
    v = jnp.mean((h - mu) ** 2, -1, keepdims=True)
    return (h - mu) * jax.lax.rsqrt(v + 1e-5) * g + b


def _dot(a, b):
    return jnp.dot(a, b, preferred_element_type=jnp.float32)


def _wg_body(g_ref, wih_t_ref, out_ref):
    wih_t = wih_t_ref[...]
    for l in range(GGC):
        out_ref[l] = _dot(g_ref[l], wih_t)


def _enc_body(x_ref, w0t, b0, g0, be0, w1t, b1, out_ref):
    h = _dot(x_ref[...], w0t[...]) + b0[...]
    h = jax.nn.relu(_ln(h, g0[...], be0[...]))
    out_ref[...] = _dot(h, w1t[...]) + b1[...]


def _gru_body(h_ref, a2_ref, wg, whht, bih, bhh, out_ref):
    a = a2_ref[0] + a2_ref[1]
    gi = _dot(a, wg[0]) + bih[...]
    gh = _dot(h_ref[...], whht[...]) + bhh[...]
    r = jax.nn.sigmoid(gi[:, :H] + gh[:, :H])
    z = jax.nn.sigmoid(gi[:, H:2 * H] + gh[:, H:2 * H])
    n = jnp.tanh(gi[:, 2 * H:] + r * gh[:, 2 * H:])
    out_ref[...] = (1.0 - z) * n + z * h_ref[...]


def _score_body(x_ref, pe_ref, thx, thp, out_ref):
    out_ref[...] = (
        lax.dot_general(thx[...], x_ref[...], (((1,), (1,)), ((), ())),
                        preferred_element_type=jnp.float32)
        + lax.dot_general(thp[...], pe_ref[...], (((1,), (1,)), ((), ())),
                          preferred_element_type=jnp.float32))


def _pool_body(s_ref, batch_ref, idx_ref, scale_ref):
    s = s_ref[...]
    brow = batch_ref[...]
    counts = []
    for b in range(NG):
        m = (brow == b).astype(jnp.float32)
        counts.append(jnp.sum(m, axis=1, keepdims=True))
    mxn = counts[0]
    for b in range(1, NG):
        mxn = jnp.maximum(mxn, counts[b])
    iota = lax.broadcasted_iota(jnp.int32, (NVT, NP), 1)
    neg = jnp.float32(-3e38)
    for b in range(NG):
        m = brow == b
        vals = jnp.where(m, s, neg)
        mx = jnp.max(vals, axis=1, keepdims=True)
        eq = vals == mx
        idx = jnp.min(jnp.where(eq, iota, jnp.int32(NP)), axis=1, keepdims=True)
        idx = jnp.minimum(idx, N - 1)
        keep = (mx >= 0.0) | (counts[b] == mxn)
        scale = keep.astype(jnp.float32) / mxn
        idx_ref[:, pl.ds(b, 1)] = idx
        scale_ref[:, pl.ds(b, 1)] = scale


def _tok_body(x_ref, pe_ref, idx_ref, scale_ref, m_int, cwt, convb, dtw, bw,
              cw_, dtwt, dtb, alog, md, moutt, lng, lnb, mwbt, mergeb,
              out_ref):
    rows = []
    for b in range(NG):
        for k in range(NVT):
            i = idx_ref[k, b]
            sc = scale_ref[k, b]
            row = jnp.concatenate(
                [x_ref[pl.ds(i, 1), :], pe_ref[pl.ds(i, 1), :]], axis=1)
            rows.append(row * sc)
    tokens = jnp.concatenate(rows, axis=0)

    xz = _dot(tokens, m_int[...])
    xi = xz[:, :DI]
    z3 = xz.reshape(NG, NVT, 2 * DI)[:, NVT - 1, DI:]

    xi4 = xi.reshape(NG, NVT, DI)
    cw = cwt[...]
    convs = []
    for t in range(NVT):
        acc = None
        for tau in range(t + 1):
            term = xi4[:, tau, :] * cw[3 - t + tau:4 - t + tau, :]
            acc = term if acc is None else acc + term
        convs.append(acc + convb[...])
    xs = jnp.concatenate([c.reshape(NG, 1, DI) for c in convs], axis=1)
    xs = xs * jax.nn.sigmoid(xs)
    xsf = xs.reshape(NG * NVT, DI)

    dt_in = _dot(xsf, dtw[...])
    bc = _dot(xsf, bw[...]).reshape(NG, NVT, DS)
    cc3 = _dot(xsf, cw_[...]).reshape(NG, NVT, DS)[:, NVT - 1, :]
    dt = jax.nn.softplus(_dot(dt_in, dtwt[...]) + dtb[...])
    dt4 = dt.reshape(NG, NVT, DI)

    a_full = -jnp.exp(alog[...])
    y_parts = []
    for dc in range(DI // DS):
        lo, hi = dc * DS, (dc + 1) * DS
        a_c = a_full[lo:hi, :]
        h = jnp.zeros((NG, DS, DS), jnp.float32)
        for t in range(NVT):
            dtc = dt4[:, t, lo:hi]
            xic = xs[:, t, lo:hi]
            da = jnp.exp(dtc[:, :, None] * a_c[None, :, :])
            dbx = (dtc * xic)[:, :, None] * bc[:, t, :][:, None, :]
            h = da * h + dbx
        y_parts.append(jnp.sum(h * cc3[:, None, :], axis=2))
    y3 = jnp.concatenate(y_parts, axis=1)
    xi3 = xs[:, NVT - 1, :]
    y3 = y3 + md[...] * xi3
    y3 = y3 * (z3 * jax.nn.sigmoid(z3))
    xm = _dot(y3, moutt[...])
    xm = _ln(xm, lng[...], lnb[...])
    out_ref[...] = _dot(xm, mwbt[...]) + mergeb[...]


def _merge_body(x1_ref, bc_ref, t16_ref, mwat, out_ref):
    oh = (bc_ref[...] == lax.broadcasted_iota(jnp.int32, (1, NG), 1))
    out_ref[...] = (_dot(x1_ref[...], mwat[...])
                    + _dot(oh.astype(jnp.float32), t16_ref[...]))


def _final_body(x_ref, a2_ref, wot, bo, bc_ref, out_ref):
    i = pl.program_id(0)
    y = _dot(x_ref[...] + a2_ref[0] + a2_ref[1], wot[...]) + bo[...]
    oh = (bc_ref[...] == lax.broadcasted_iota(jnp.int32, (1, NG), 1))
    part = lax.dot_general(oh.astype(jnp.float32), y, (((0,), (0,)), ((), ())),
                           preferred_element_type=jnp.float32)

    @pl.when(i == 0)
    def _():
        out_ref[...] = jnp.zeros_like(out_ref)

    out_ref[...] += part


def _dec_body(s_ref, w0t, b0, g0, be0, w1t, b1, out_ref):
    h = _dot(s_ref[...], w0t[...]) + b0[...]
    h = jax.nn.relu(_ln(h, g0[...], be0[...]))
    out_ref[...] = _dot(h, w1t[...]) + b1[...]


def _row_spec(r, c):
    return pl.BlockSpec((r, c), lambda i: (i, 0))


def _full_spec(shape):
    nd = len(shape)
    return pl.BlockSpec(shape, lambda i, _n=nd: (0,) * _n)


def _call_rows(body, out_shape, row_ins, full_ins, out_full=False):
    in_specs = ([_row_spec(FB, c) for _, c in row_ins]
                + [_full_spec(a.shape) for a in full_ins])
    if out_full:
        out_spec = _full_spec(out_shape)
    else:
        out_spec = _row_spec(FB, out_shape[-1])
    return pl.pallas_call(
        body,
        grid=(GRID,),
        in_specs=in_specs,
        out_specs=out_spec,
        out_shape=jax.ShapeDtypeStruct(out_shape, jnp.float32),
    )(*[a for a, _ in row_ins], *full_ins)


def kernel(x, edge_index, batch, params):
    p = params
    f32 = jnp.float32

    src = edge_index[0]
    dst = edge_index[1]
    src_p = jnp.concatenate(
        [src, jnp.zeros((EPAD - E,), jnp.int32)]).reshape(NW, NCH, CH)
    dst_p = jnp.concatenate(
        [dst, jnp.full((EPAD - E,), NP - 1, jnp.int32)]).reshape(NW, NCH, CH)
    zrows = jnp.zeros((SLAB, H), f32)
    x = jnp.concatenate([x, jnp.zeros((NP - N, x.shape[1]), f32)])
    batch_p = jnp.concatenate([batch, jnp.full((NP - N,), NG, jnp.int32)])
    bcol = batch_p.reshape(NP, 1)
    brow = batch_p.reshape(1, NP)

    def row(v):
        return v.reshape(1, -1)

    wg = pl.pallas_call(
        _wg_body,
        out_shape=jax.ShapeDtypeStruct((GGC, H, 3 * H), f32),
    )(p['ggc_w'], p['gru_Wih'].T)

    x1 = _call_rows(
        _enc_body, (NP, H), [(x, H)],
        [p['enc_W0'].T, row(p['enc_b0']), row(p['enc_g0']), row(p['enc_be0']),
         p['enc_W1'].T, row(p['enc_b1'])])

    whht = p['gru_Whh'].T
    bih = row(p['gru_bih'])
    bhh = row(p['gru_bhh'])
    h = x1
    for l in range(GGC):
        a2 = _sc_agg(h, src_p, dst_p, zrows)
        h = pl.pallas_call(
            _gru_body,
            grid=(GRID,),
            in_specs=[
                _row_spec(FB, H),
                pl.BlockSpec((NC, FB, H), lambda i: (0, i, 0)),
                pl.BlockSpec((1, H, 3 * H), lambda i, _l=l: (_l, 0, 0)),
                _full_spec(whht.shape),
                _full_spec(bih.shape),
                _full_spec(bhh.shape),
            ],
            out_specs=_row_spec(FB, H),
            out_shape=jax.ShapeDtypeStruct((NP, H), f32),
        )(h, a2, wg, whht, bih, bhh)
    pe = h

    xc = x1
    for i in range(NL):
        thx = p['theta'][i][:, :H]
        thp = p['theta'][i][:, H:]
        s_t = pl.pallas_call(
            _score_body,
            grid=(GRID,),
            in_specs=[_row_spec(FB, H), _row_spec(FB, H),
                      _full_spec(thx.shape), _full_spec(thp.shape)],
            out_specs=pl.BlockSpec((NVT, FB), lambda i: (0, i)),
            out_shape=jax.ShapeDtypeStruct((NVT, NP), f32),
        )(xc, pe, thx, thp)

        idxs, scales = pl.pallas_call(
            _pool_body,
            out_shape=(jax.ShapeDtypeStruct((NVT, NG), jnp.int32),
                       jax.ShapeDtypeStruct((NVT, NG), f32)),
        )(s_t, brow)

        mw = [p['m_in'][i].T, p['m_conv_w'][i].T, row(p['m_conv_b'][i]),
              p['m_xproj'][i][:DTR].T, p['m_xproj'][i][DTR:DTR + DS].T,
              p['m_xproj'][i][DTR + DS:].T, p['m_dt_w'][i].T,
              row(p['m_dt_b'][i]), p['m_Alog'][i], row(p['m_D'][i]),
              p['m_out'][i].T, row(p['ln_m_g']), row(p['ln_m_b']),
              p['merge_W'][i][:, H:].T, row(p['merge_b'][i])]

        def _fs(a):
            return pl.BlockSpec(a.shape, lambda *_, _n=a.ndim: (0,) * _n)

        t16 = pl.pallas_call(
            _tok_body,
            in_specs=([_fs(xc), _fs(pe),
                       pl.BlockSpec(memory_space=pltpu.SMEM),
                       pl.BlockSpec(memory_space=pltpu.SMEM)]
                      + [_fs(a) for a in mw]),
            out_specs=pl.BlockSpec((NG, H), lambda *_: (0, 0)),
            out_shape=jax.ShapeDtypeStruct((NG, H), f32),
        )(xc, pe, idxs, scales, *mw)

        xc = _call_rows(
            _merge_body, (NP, H), [(x1, H), (bcol, 1)],
            [t16, p['merge_W'][i][:, :H].T])

    a2 = _sc_agg(xc, src_p, dst_p, zrows)
    seg = pl.pallas_call(
        _final_body,
        grid=(GRID,),
        in_specs=[
            _row_spec(FB, H),
            pl.BlockSpec((NC, FB, H), lambda i: (0, i, 0)),
            _full_spec((H, H)),
            _full_spec((1, H)),
            _row_spec(FB, 1),
        ],
        out_specs=pl.BlockSpec((NG, H), lambda i: (0, 0)),
        out_shape=jax.ShapeDtypeStruct((NG, H), f32),
    )(xc, a2, p['outgin_W'].T, row(p['outgin_b']), bcol)

    out = pl.pallas_call(
        _dec_body,
        out_shape=jax.ShapeDtypeStruct((NG, H), f32),
    )(seg, p['dec_W0'].T, row(p['dec_b0']), row(p['dec_g0']),
      row(p['dec_be0']), p['dec_W1'].T, row(p['dec_b1']))
    return out


# speedup vs baseline: 1.2169x; 1.0050x over previous
import functools

import jax
import jax.numpy as jnp
from jax import lax
from jax.experimental import pallas as pl
from jax.experimental.pallas import tpu as pltpu
from jax.experimental.pallas import tpu_sc as plsc

N = 10000
NP = 10240
E = 320000
H = 128
NG = 16
NVT = 4
DM = 256
DI = 512
DS = 128
DTR = 16
GGC = 8
NL = 2

NC = 2
NSUB = 16
NW = NC * NSUB
CH = 32
NCH = 320
EPW = NCH * CH
EPAD = EPW * NW
NPAD = NP
SLAB = NPAD // NSUB
PH = 8
CPP = NCH // PH

FB = 1024
GRID = NP // FB


def _sc_agg_body(h_hbm, srcs, dsts, zrows, out, src_v, dst_v, r0, r1, r2, r3,
                 r4, r5, r6, r7, a_sh, sem0, sem1, sem2, sem3, sem4, sem5,
                 sem6, sem7, ssem0, ssem1, ssem2, ssem3, ssem4, ssem5, ssem6,
                 ssem7):
    c = lax.axis_index("c")
    s = lax.axis_index("s")
    wid = c * NSUB + s

    pltpu.sync_copy(zrows, a_sh.at[pl.ds(s * SLAB, SLAB)])
    plsc.subcore_barrier()

    def gather(j, buf, sem):
        return pltpu.async_copy(h_hbm.at[src_v.at[j]], buf, sem)

    def wait(j, buf, sem):
        pltpu.make_async_copy(h_hbm.at[src_v.at[j]], buf, sem).wait()

    def scat_start(buf, j, sem):
        return pltpu.async_copy(buf, a_sh.at[dst_v.at[j]], sem, add=True)

    def scat_wait(buf, j, sem):
        pltpu.make_async_copy(buf, a_sh.at[dst_v.at[j]], sem).wait()

    bufs = (r0, r1, r2, r3, r4, r5, r6, r7)
    gsems = (sem0, sem1, sem2, sem3, sem4, sem5, sem6, sem7)
    ssems = (ssem0, ssem1, ssem2, ssem3, ssem4, ssem5, ssem6, ssem7)
    for p in range(PH):
        pltpu.sync_copy(srcs.at[wid, pl.ds(p * CPP, CPP)], src_v)
        pltpu.sync_copy(dsts.at[wid, pl.ds(p * CPP, CPP)], dst_v)
        for k in range(8):
            gather(k, bufs[k], gsems[k])

        def body(g, carry):
            for k in range(8):
                j = 8 * g + k
                wait(j, bufs[k], gsems[k])
                scat_start(bufs[k], j, ssems[k])

            @pl.when(g < CPP // 8 - 1)
            def _():
                for k in range(8):
                    j = 8 * g + k
                    scat_wait(bufs[k], j, ssems[k])
                    gather(j + 8, bufs[k], gsems[k])
            return carry

        lax.fori_loop(0, CPP // 8, body, 0)
        for k in range(8):
            scat_wait(bufs[k], CPP - 8 + k, ssems[k])
    plsc.subcore_barrier()

    pltpu.sync_copy(a_sh.at[pl.ds(s * SLAB, SLAB)],
                    out.at[c, pl.ds(s * SLAB, SLAB)])


@functools.cache
def _make_sc_agg():
    return functools.partial(
        pl.kernel,
        out_type=jax.ShapeDtypeStruct((NC, NPAD, H), jnp.float32),
        mesh=plsc.VectorSubcoreMesh(core_axis_name="c", subcore_axis_name="s",
                                    num_cores=NC, num_subcores=NSUB),
        scratch_types=[
            pltpu.VMEM((CPP, CH), jnp.int32),
            pltpu.VMEM((CPP, CH), jnp.int32),
        ] + [pltpu.VMEM((CH, H), jnp.float32)] * 8 + [
            pltpu.VMEM_SHARED((NPAD, H), jnp.float32),
        ] + [pltpu.SemaphoreType.DMA] * 16,
    )(_sc_agg_body)


def _sc_agg(h, src_p, dst_p, zrows):
    return _make_sc_agg()(h, src_p, dst_p, zrows)


def _ln(h, g, b):
    mu = jnp.mean(h, -1, keepdims=True)
    v = jnp.mean((h - mu) ** 2, -1, keepdims=True)
    return (h - mu) * jax.lax.rsqrt(v + 1e-5) * g + b


def _dot(a, b):
    return jnp.dot(a, b, preferred_element_type=jnp.float32)


def _wg_body(g_ref, wih_t_ref, out_ref):
    wih_t = wih_t_ref[...]
    for l in range(GGC):
        out_ref[l] = _dot(g_ref[l], wih_t)


def _enc_body(x_ref, w0t, b0, g0, be0, w1t, b1, out_ref):
    h = _dot(x_ref[...], w0t[...]) + b0[...]
    h = jax.nn.relu(_ln(h, g0[...], be0[...]))
    out_ref[...] = _dot(h, w1t[...]) + b1[...]


def _gru_body(h_ref, a2_ref, wg, whht, bih, bhh, out_ref):
    a = a2_ref[0] + a2_ref[1]
    gi = _dot(a, wg[0]) + bih[...]
    gh = _dot(h_ref[...], whht[...]) + bhh[...]
    r = jax.nn.sigmoid(gi[:, :H] + gh[:, :H])
    z = jax.nn.sigmoid(gi[:, H:2 * H] + gh[:, H:2 * H])
    n = jnp.tanh(gi[:, 2 * H:] + r * gh[:, 2 * H:])
    out_ref[...] = (1.0 - z) * n + z * h_ref[...]


def _score_body(x_ref, pe_ref, thx, thp, out_ref):
    out_ref[...] = (
        lax.dot_general(thx[...], x_ref[...], (((1,), (1,)), ((), ())),
                        preferred_element_type=jnp.float32)
        + lax.dot_general(thp[...], pe_ref[...], (((1,), (1,)), ((), ())),
                          preferred_element_type=jnp.float32))


def _pool_body(s_ref, batch_ref, idx_ref, scale_ref):
    s = s_ref[...]
    brow = batch_ref[...]
    counts = []
    for b in range(NG):
        m = (brow == b).astype(jnp.float32)
        counts.append(jnp.sum(m, axis=1, keepdims=True))
    mxn = counts[0]
    for b in range(1, NG):
        mxn = jnp.maximum(mxn, counts[b])
    iota = lax.broadcasted_iota(jnp.int32, (NVT, NP), 1)
    neg = jnp.float32(-3e38)
    for b in range(NG):
        m = brow == b
        vals = jnp.where(m, s, neg)
        mx = jnp.max(vals, axis=1, keepdims=True)
        eq = vals == mx
        idx = jnp.min(jnp.where(eq, iota, jnp.int32(NP)), axis=1, keepdims=True)
        idx = jnp.minimum(idx, N - 1)
        keep = (mx >= 0.0) | (counts[b] == mxn)
        scale = keep.astype(jnp.float32) / mxn
        idx_ref[:, pl.ds(b, 1)] = idx
        scale_ref[:, pl.ds(b, 1)] = scale


def _tok_body(x_ref, pe_ref, idx_ref, scale_ref, m_int, cwt, convb, dtw, bw,
              cw_, dtwt, dtb, alog, md, moutt, lng, lnb, mwbt, mergeb,
              out_ref):
    rows = []
    for b in range(NG):
        for k in range(NVT):
            i = idx_ref[k, b]
            sc = scale_ref[k, b]
            row = jnp.concatenate(
                [x_ref[pl.ds(i, 1), :], pe_ref[pl.ds(i, 1), :]], axis=1)
            rows.append(row * sc)
    tokens = jnp.concatenate(rows, axis=0)

    xz = _dot(tokens, m_int[...])
    xi = xz[:, :DI]
    z3 = xz.reshape(NG, NVT, 2 * DI)[:, NVT - 1, DI:]

    xi4 = xi.reshape(NG, NVT, DI)
    cw = cwt[...]
    convs = []
    for t in range(NVT):
        acc = None
        for tau in range(t + 1):
            term = xi4[:, tau, :] * cw[3 - t + tau:4 - t + tau, :]
            acc = term if acc is None else acc + term
        convs.append(acc + convb[...])
    xs = jnp.concatenate([c.reshape(NG, 1, DI) for c in convs], axis=1)
    xs = xs * jax.nn.sigmoid(xs)
    xsf = xs.reshape(NG * NVT, DI)

    dt_in = _dot(xsf, dtw[...])
    bc = _dot(xsf, bw[...]).reshape(NG, NVT, DS)
    cc3 = _dot(xsf, cw_[...]).reshape(NG, NVT, DS)[:, NVT - 1, :]
    dt = jax.nn.softplus(_dot(dt_in, dtwt[...]) + dtb[...])
    dt4 = dt.reshape(NG, NVT, DI)

    a_full = -jnp.exp(alog[...])
    y_parts = []
    for dc in range(DI // DS):
        lo, hi = dc * DS, (dc + 1) * DS
        a_c = a_full[lo:hi, :]
        h = jnp.zeros((NG, DS, DS), jnp.float32)
        for t in range(NVT):
            dtc = dt4[:, t, lo:hi]
            xic = xs[:, t, lo:hi]
            da = jnp.exp(dtc[:, :, None] * a_c[None, :, :])
            dbx = (dtc * xic)[:, :, None] * bc[:, t, :][:, None, :]
            h = da * h + dbx
        y_parts.append(jnp.sum(h * cc3[:, None, :], axis=2))
    y3 = jnp.concatenate(y_parts, axis=1)
    xi3 = xs[:, NVT - 1, :]
    y3 = y3 + md[...] * xi3
    y3 = y3 * (z3 * jax.nn.sigmoid(z3))
    xm = _dot(y3, moutt[...])
    xm = _ln(xm, lng[...], lnb[...])
    out_ref[...] = _dot(xm, mwbt[...]) + mergeb[...]


def _merge_body(x1_ref, bc_ref, t16_ref, mwat, out_ref):
    oh = (bc_ref[...] == lax.broadcasted_iota(jnp.int32, (1, NG), 1))
    out_ref[...] = (_dot(x1_ref[...], mwat[...])
                    + _dot(oh.astype(jnp.float32), t16_ref[...]))


def _final_body(x_ref, a2_ref, wot, bo, bc_ref, out_ref):
    i = pl.program_id(0)
    y = _dot(x_ref[...] + a2_ref[0] + a2_ref[1], wot[...]) + bo[...]
    oh = (bc_ref[...] == lax.broadcasted_iota(jnp.int32, (1, NG), 1))
    part = lax.dot_general(oh.astype(jnp.float32), y, (((0,), (0,)), ((), ())),
                           preferred_element_type=jnp.float32)

    @pl.when(i == 0)
    def _():
        out_ref[...] = jnp.zeros_like(out_ref)

    out_ref[...] += part


def _dec_body(s_ref, w0t, b0, g0, be0, w1t, b1, out_ref):
    h = _dot(s_ref[...], w0t[...]) + b0[...]
    h = jax.nn.relu(_ln(h, g0[...], be0[...]))
    out_ref[...] = _dot(h, w1t[...]) + b1[...]


def _row_spec(r, c):
    return pl.BlockSpec((r, c), lambda i: (i, 0))


def _full_spec(shape):
    nd = len(shape)
    return pl.BlockSpec(shape, lambda i, _n=nd: (0,) * _n)


def _call_rows(body, out_shape, row_ins, full_ins, out_full=False):
    in_specs = ([_row_spec(FB, c) for _, c in row_ins]
                + [_full_spec(a.shape) for a in full_ins])
    if out_full:
        out_spec = _full_spec(out_shape)
    else:
        out_spec = _row_spec(FB, out_shape[-1])
    return pl.pallas_call(
        body,
        grid=(GRID,),
        in_specs=in_specs,
        out_specs=out_spec,
        out_shape=jax.ShapeDtypeStruct(out_shape, jnp.float32),
    )(*[a for a, _ in row_ins], *full_ins)


def kernel(x, edge_index, batch, params):
    p = params
    f32 = jnp.float32

    src = edge_index[0]
    dst = edge_index[1]
    src_p = jnp.concatenate(
        [src, jnp.zeros((EPAD - E,), jnp.int32)]).reshape(NW, NCH, CH)
    dst_p = jnp.concatenate(
        [dst, jnp.full((EPAD - E,), NP - 1, jnp.int32)]).reshape(NW, NCH, CH)
    zrows = jnp.zeros((SLAB, H), f32)
    x = jnp.concatenate([x, jnp.zeros((NP - N, x.shape[1]), f32)])
    batch_p = jnp.concatenate([batch, jnp.full((NP - N,), NG, jnp.int32)])
    bcol = batch_p.reshape(NP, 1)
    brow = batch_p.reshape(1, NP)

    def row(v):
        return v.reshape(1, -1)

    wg = pl.pallas_call(
        _wg_body,
        out_shape=jax.ShapeDtypeStruct((GGC, H, 3 * H), f32),
    )(p['ggc_w'], p['gru_Wih'].T)

    x1 = _call_rows(
        _enc_body, (NP, H), [(x, H)],
        [p['enc_W0'].T, row(p['enc_b0']), row(p['enc_g0']), row(p['enc_be0']),
         p['enc_W1'].T, row(p['enc_b1'])])

    whht = p['gru_Whh'].T
    bih = row(p['gru_bih'])
    bhh = row(p['gru_bhh'])
    h = x1
    for l in range(GGC):
        a2 = _sc_agg(h, src_p, dst_p, zrows)
        h = pl.pallas_call(
            _gru_body,
            grid=(GRID,),
            in_specs=[
                _row_spec(FB, H),
                pl.BlockSpec((NC, FB, H), lambda i: (0, i, 0)),
                pl.BlockSpec((1, H, 3 * H), lambda i, _l=l: (_l, 0, 0)),
                _full_spec(whht.shape),
                _full_spec(bih.shape),
                _full_spec(bhh.shape),
            ],
            out_specs=_row_spec(FB, H),
            out_shape=jax.ShapeDtypeStruct((NP, H), f32),
        )(h, a2, wg, whht, bih, bhh)
    pe = h

    xc = x1
    for i in range(NL):
        thx = p['theta'][i][:, :H]
        thp = p['theta'][i][:, H:]
        s_t = pl.pallas_call(
            _score_body,
            grid=(GRID,),
            in_specs=[_row_spec(FB, H), _row_spec(FB, H),
                      _full_spec(thx.shape), _full_spec(thp.shape)],
            out_specs=pl.BlockSpec((NVT, FB), lambda i: (0, i)),
            out_shape=jax.ShapeDtypeStruct((NVT, NP), f32),
        )(xc, pe, thx, thp)

        idxs, scales = pl.pallas_call(
            _pool_body,
            out_shape=(jax.ShapeDtypeStruct((NVT, NG), jnp.int32),
                       jax.ShapeDtypeStruct((NVT, NG), f32)),
        )(s_t, brow)

        mw = [p['m_in'][i].T, p['m_conv_w'][i].T, row(p['m_conv_b'][i]),
              p['m_xproj'][i][:DTR].T, p['m_xproj'][i][DTR:DTR + DS].T,
              p['m_xproj'][i][DTR + DS:].T, p['m_dt_w'][i].T,
              row(p['m_dt_b'][i]), p['m_Alog'][i], row(p['m_D'][i]),
              p['m_out'][i].T, row(p['ln_m_g']), row(p['ln_m_b']),
              p['merge_W'][i][:, H:].T, row(p['merge_b'][i])]

        def _fs(a):
            return pl.BlockSpec(a.shape, lambda *_, _n=a.ndim: (0,) * _n)

        t16 = pl.pallas_call(
            _tok_body,
            in_specs=([_fs(xc), _fs(pe),
                       pl.BlockSpec(memory_space=pltpu.SMEM),
                       pl.BlockSpec(memory_space=pltpu.SMEM)]
                      + [_fs(a) for a in mw]),
            out_specs=pl.BlockSpec((NG, H), lambda *_: (0, 0)),
            out_shape=jax.ShapeDtypeStruct((NG, H), f32),
        )(xc, pe, idxs, scales, *mw)

        xc = _call_rows(
            _merge_body, (NP, H), [(x1, H), (bcol, 1)],
            [t16, p['merge_W'][i][:, :H].T])

    a2 = _sc_agg(xc, src_p, dst_p, zrows)
    seg = pl.pallas_call(
        _final_body,
        grid=(GRID,),
        in_specs=[
            _row_spec(FB, H),
            pl.BlockSpec((NC, FB, H), lambda i: (0, i, 0)),
            _full_spec((H, H)),
            _full_spec((1, H)),
            _row_spec(FB, 1),
        ],
        out_specs=pl.BlockSpec((NG, H), lambda i: (0, 0)),
        out_shape=jax.ShapeDtypeStruct((NG, H), f32),
    )(xc, a2, p['outgin_W'].T, row(p['outgin_b']), bcol)

    out = pl.pallas_call(
        _dec_body,
        out_shape=jax.ShapeDtypeStruct((NG, H), f32),
    )(seg, p['dec_W0'].T, row(p['dec_b0']), row(p['dec_g0']),
      row(p['dec_be0']), p['dec_W1'].T, row(p['dec_b1']))
    return out
